# baseline (speedup 1.0000x reference)
import functools

import jax
import jax.numpy as jnp
from jax import lax
from jax.experimental import pallas as pl
from jax.experimental.pallas import tpu as pltpu
from jax.experimental.pallas import tpu_sc as plsc


def _make_sc_gather(B, b_per_w, NC):
    mesh = plsc.VectorSubcoreMesh(core_axis_name="c", subcore_axis_name="s")

    @functools.partial(
        pl.kernel,
        mesh=mesh,
        out_type=jax.ShapeDtypeStruct((B, 128), jnp.float32),
        scratch_types=[
            pltpu.VMEM((b_per_w,), jnp.int32),
            pltpu.VMEM((b_per_w, 128), jnp.float32),
            pltpu.SemaphoreType.DMA,
        ],
    )
    def sc_gather(table_hbm, idx_hbm, out_hbm, idx_v, rows_v, sem):
        wid = lax.axis_index("s") * NC + lax.axis_index("c")
        base = wid * b_per_w
        pltpu.sync_copy(idx_hbm.at[pl.ds(base, b_per_w)], idx_v)
        pltpu.async_copy(table_hbm.at[idx_v], rows_v, sem).wait()
        pltpu.sync_copy(rows_v, out_hbm.at[pl.ds(base, b_per_w)])

    return sc_gather


def _tower_body(x_ref, mv4_ref, off_ref, w1_ref, b1_ref, w2_ref, b2_ref,
                w3s_ref, w3b_ref, b3_ref, w4_ref, b4_ref, out_ref):
    h = jnp.maximum(
        jnp.dot(x_ref[...], w1_ref[...], preferred_element_type=jnp.float32)
        + b1_ref[...], 0.0)
    cv = jnp.dot(h, w2_ref[...], preferred_element_type=jnp.float32) + b2_ref[...]
    seg = lax.broadcasted_iota(jnp.int32, mv4_ref.shape, 1) // 32
    mv4m = jnp.where(seg == off_ref[...], mv4_ref[...], 0.0)
    h2 = jnp.maximum(
        jnp.dot(mv4m, w3s_ref[...], preferred_element_type=jnp.float32)
        + jnp.dot(cv, w3b_ref[...], preferred_element_type=jnp.float32)
        + b3_ref[...], 0.0)
    out_ref[...] = (
        jnp.dot(h2, w4_ref[...], preferred_element_type=jnp.float32) + b4_ref[...])


def kernel(movie_ids, content_features, embed_table, W1, b1, W2, b2, W3, b3, W4, b4):
    B, NC_FEAT = content_features.shape
    V, D = embed_table.shape
    H1 = W1.shape[1]
    H2 = W2.shape[1]
    H3 = W3.shape[1]
    OUT = W4.shape[1]
    PACK = 128 // D

    info = plsc.get_sparse_core_info()
    NW = info.num_cores * info.num_subcores
    b_per_w = B // NW

    ids = movie_ids.astype(jnp.int32)
    idx4 = ids // PACK
    off = (ids % PACK).reshape(B, 1)
    table4 = embed_table.reshape(V // PACK, 128)

    mv4 = _make_sc_gather(B, b_per_w, info.num_cores)(table4, idx4)

    W3s = jnp.tile(W3[:D], (PACK, 1))
    W3b = W3[D:]

    BLK = 2048
    grid = (B // BLK,)

    out = pl.pallas_call(
        _tower_body,
        grid=grid,
        in_specs=[
            pl.BlockSpec((BLK, NC_FEAT), lambda i: (i, 0)),
            pl.BlockSpec((BLK, 128), lambda i: (i, 0)),
            pl.BlockSpec((BLK, 1), lambda i: (i, 0)),
            pl.BlockSpec((NC_FEAT, H1), lambda i: (0, 0)),
            pl.BlockSpec((1, H1), lambda i: (0, 0)),
            pl.BlockSpec((H1, H2), lambda i: (0, 0)),
            pl.BlockSpec((1, H2), lambda i: (0, 0)),
            pl.BlockSpec((128, H3), lambda i: (0, 0)),
            pl.BlockSpec((H2, H3), lambda i: (0, 0)),
            pl.BlockSpec((1, H3), lambda i: (0, 0)),
            pl.BlockSpec((H3, OUT), lambda i: (0, 0)),
            pl.BlockSpec((1, OUT), lambda i: (0, 0)),
        ],
        out_specs=pl.BlockSpec((BLK, OUT), lambda i: (i, 0)),
        out_shape=jax.ShapeDtypeStruct((B, OUT), jnp.float32),
    )(content_features, mv4, off, W1, b1.reshape(1, H1), W2, b2.reshape(1, H2),
      W3s, W3b, b3.reshape(1, H3), W4, b4.reshape(1, OUT))
    return out


# speedup vs baseline: 1.6404x; 1.6404x over previous
import functools

import jax
import jax.numpy as jnp
from jax import lax
from jax.experimental import pallas as pl
from jax.experimental.pallas import tpu as pltpu
from jax.experimental.pallas import tpu_sc as plsc


def _make_sc_gather(D, B, b_per_w, NC):
    mesh = plsc.VectorSubcoreMesh(core_axis_name="c", subcore_axis_name="s")

    @functools.partial(
        pl.kernel,
        mesh=mesh,
        compiler_params=pltpu.CompilerParams(needs_layout_passes=False),
        out_type=jax.ShapeDtypeStruct((B, D), jnp.float32),
        scratch_types=[
            pltpu.VMEM((b_per_w,), jnp.int32),
            pltpu.VMEM((b_per_w, D), jnp.float32),
            pltpu.SemaphoreType.DMA,
        ],
    )
    def sc_gather(table_hbm, ids_hbm, out_hbm, ids_v, out_v, sem):
        wid = lax.axis_index("s") * NC + lax.axis_index("c")
        base = wid * b_per_w
        pltpu.sync_copy(ids_hbm.at[pl.ds(base, b_per_w)], ids_v)

        def fire(g, _):
            row0 = g * 16
            v16 = ids_v[pl.ds(row0, 16)]
            for k in range(16):
                pltpu.make_async_copy(
                    table_hbm.at[v16[k]], out_v.at[row0 + k], sem
                ).start()
            return 0

        lax.fori_loop(0, b_per_w // 16, fire, 0)
        pltpu.make_async_copy(table_hbm.at[pl.ds(0, b_per_w)], out_v, sem).wait()
        pltpu.sync_copy(out_v, out_hbm.at[pl.ds(base, b_per_w)])

    return sc_gather


def _tower_body(x_ref, mv_ref, w1_ref, b1_ref, w2_ref, b2_ref,
                w3a_ref, w3b_ref, b3_ref, w4_ref, b4_ref, out_ref):
    h = jnp.maximum(
        jnp.dot(x_ref[...], w1_ref[...], preferred_element_type=jnp.float32)
        + b1_ref[...], 0.0)
    cv = jnp.dot(h, w2_ref[...], preferred_element_type=jnp.float32) + b2_ref[...]
    h2 = jnp.maximum(
        jnp.dot(mv_ref[...], w3a_ref[...], preferred_element_type=jnp.float32)
        + jnp.dot(cv, w3b_ref[...], preferred_element_type=jnp.float32)
        + b3_ref[...], 0.0)
    out_ref[...] = (
        jnp.dot(h2, w4_ref[...], preferred_element_type=jnp.float32) + b4_ref[...])


def kernel(movie_ids, content_features, embed_table, W1, b1, W2, b2, W3, b3, W4, b4):
    B, NC_FEAT = content_features.shape
    V, D = embed_table.shape
    H1 = W1.shape[1]
    H2 = W2.shape[1]
    H3 = W3.shape[1]
    OUT = W4.shape[1]

    info = plsc.get_sparse_core_info()
    NW = info.num_cores * info.num_subcores
    b_per_w = B // NW

    ids = movie_ids.astype(jnp.int32)

    mv = _make_sc_gather(D, B, b_per_w, info.num_cores)(embed_table, ids)

    W3a = W3[:D]
    W3b = W3[D:]

    BLK = 2048
    grid = (B // BLK,)

    out = pl.pallas_call(
        _tower_body,
        grid=grid,
        in_specs=[
            pl.BlockSpec((BLK, NC_FEAT), lambda i: (i, 0)),
            pl.BlockSpec((BLK, D), lambda i: (i, 0)),
            pl.BlockSpec((NC_FEAT, H1), lambda i: (0, 0)),
            pl.BlockSpec((1, H1), lambda i: (0, 0)),
            pl.BlockSpec((H1, H2), lambda i: (0, 0)),
            pl.BlockSpec((1, H2), lambda i: (0, 0)),
            pl.BlockSpec((D, H3), lambda i: (0, 0)),
            pl.BlockSpec((H2, H3), lambda i: (0, 0)),
            pl.BlockSpec((1, H3), lambda i: (0, 0)),
            pl.BlockSpec((H3, OUT), lambda i: (0, 0)),
            pl.BlockSpec((1, OUT), lambda i: (0, 0)),
        ],
        out_specs=pl.BlockSpec((BLK, OUT), lambda i: (i, 0)),
        out_shape=jax.ShapeDtypeStruct((B, OUT), jnp.float32),
    )(content_features, mv, W1, b1.reshape(1, H1), W2, b2.reshape(1, H2),
      W3a, W3b, b3.reshape(1, H3), W4, b4.reshape(1, OUT))
    return out


# speedup vs baseline: 1.8758x; 1.1435x over previous
import functools

import jax
import jax.numpy as jnp
from jax import lax
from jax.experimental import pallas as pl
from jax.experimental.pallas import tpu as pltpu
from jax.experimental.pallas import tpu_sc as plsc

_LBLK = 8192


def _transpose_body(xt_ref, eye_ref, out_ref):
    out_ref[...] = lax.dot_general(
        xt_ref[...], eye_ref[...], (((0,), (0,)), ((), ())),
        preferred_element_type=jnp.float32)


def _make_sc_gather(D, B, b_per_w, NC):
    mesh = plsc.VectorSubcoreMesh(core_axis_name="c", subcore_axis_name="s")

    @functools.partial(
        pl.kernel,
        mesh=mesh,
        compiler_params=pltpu.CompilerParams(needs_layout_passes=False),
        out_type=jax.ShapeDtypeStruct((B, D), jnp.float32),
        scratch_types=[
            pltpu.VMEM((b_per_w,), jnp.int32),
            pltpu.VMEM((b_per_w, D), jnp.float32),
            pltpu.SemaphoreType.DMA,
        ],
    )
    def sc_gather(table_hbm, ids_hbm, out_hbm, ids_v, out_v, sem):
        wid = lax.axis_index("s") * NC + lax.axis_index("c")
        base = wid * b_per_w
        pltpu.sync_copy(ids_hbm.at[pl.ds(base, b_per_w)], ids_v)

        def fire(g, _):
            row0 = g * 16
            v16 = ids_v[pl.ds(row0, 16)]
            for k in range(16):
                pltpu.make_async_copy(
                    table_hbm.at[v16[k]], out_v.at[row0 + k], sem
                ).start()
            return 0

        lax.fori_loop(0, b_per_w // 16, fire, 0)
        pltpu.make_async_copy(table_hbm.at[pl.ds(0, b_per_w)], out_v, sem).wait()
        pltpu.sync_copy(out_v, out_hbm.at[pl.ds(base, b_per_w)])

    return sc_gather


def _tower_body(x_ref, mv_ref, w1_ref, b1_ref, w2_ref, b2_ref,
                w3a_ref, w3b_ref, b3_ref, w4_ref, b4_ref, out_ref):
    h = jnp.maximum(
        jnp.dot(x_ref[...], w1_ref[...], preferred_element_type=jnp.float32)
        + b1_ref[...], 0.0)
    cv = jnp.dot(h, w2_ref[...], preferred_element_type=jnp.float32) + b2_ref[...]
    h2 = jnp.maximum(
        jnp.dot(mv_ref[...], w3a_ref[...], preferred_element_type=jnp.float32)
        + jnp.dot(cv, w3b_ref[...], preferred_element_type=jnp.float32)
        + b3_ref[...], 0.0)
    out_ref[...] = (
        jnp.dot(h2, w4_ref[...], preferred_element_type=jnp.float32) + b4_ref[...])


def kernel(movie_ids, content_features, embed_table, W1, b1, W2, b2, W3, b3, W4, b4):
    B, NC_FEAT = content_features.shape
    V, D = embed_table.shape
    H1 = W1.shape[1]
    H2 = W2.shape[1]
    H3 = W3.shape[1]
    OUT = W4.shape[1]

    info = plsc.get_sparse_core_info()
    NW = info.num_cores * info.num_subcores
    b_per_w = B // NW

    ids = movie_ids.astype(jnp.int32)

    n_lblk = (V + _LBLK - 1) // _LBLK
    table_rm = pl.pallas_call(
        _transpose_body,
        grid=(n_lblk,),
        in_specs=[
            pl.BlockSpec((D, _LBLK), lambda i: (0, i)),
            pl.BlockSpec((D, D), lambda i: (0, 0)),
        ],
        out_specs=pl.BlockSpec((_LBLK, D), lambda i: (i, 0)),
        out_shape=jax.ShapeDtypeStruct((V, D), jnp.float32),
    )(embed_table.T, jnp.eye(D, dtype=jnp.float32))

    mv = _make_sc_gather(D, B, b_per_w, info.num_cores)(table_rm, ids)

    W3a = W3[:D]
    W3b = W3[D:]

    BLK = 2048
    grid = (B // BLK,)

    out = pl.pallas_call(
        _tower_body,
        grid=grid,
        in_specs=[
            pl.BlockSpec((BLK, NC_FEAT), lambda i: (i, 0)),
            pl.BlockSpec((BLK, D), lambda i: (i, 0)),
            pl.BlockSpec((NC_FEAT, H1), lambda i: (0, 0)),
            pl.BlockSpec((1, H1), lambda i: (0, 0)),
            pl.BlockSpec((H1, H2), lambda i: (0, 0)),
            pl.BlockSpec((1, H2), lambda i: (0, 0)),
            pl.BlockSpec((D, H3), lambda i: (0, 0)),
            pl.BlockSpec((H2, H3), lambda i: (0, 0)),
            pl.BlockSpec((1, H3), lambda i: (0, 0)),
            pl.BlockSpec((H3, OUT), lambda i: (0, 0)),
            pl.BlockSpec((1, OUT), lambda i: (0, 0)),
        ],
        out_specs=pl.BlockSpec((BLK, OUT), lambda i: (i, 0)),
        out_shape=jax.ShapeDtypeStruct((B, OUT), jnp.float32),
    )(content_features, mv, W1, b1.reshape(1, H1), W2, b2.reshape(1, H2),
      W3a, W3b, b3.reshape(1, H3), W4, b4.reshape(1, OUT))
    return out
